# Initial kernel scaffold; baseline (speedup 1.0000x reference)
import jax
import jax.numpy as jnp
from jax.experimental import pallas as pl


def kernel(x_in, norm1_w, norm2_w, in_proj_w, conv_w, conv_b, x_proj_w, dt_proj_w, dt_proj_b, A_log, D_param, out_proj_w, fc1_w, fc1_b, dw_w, dw_b, fc2_w, fc2_b):
    raise NotImplementedError("write your pallas kernel here")


# speedup vs baseline: 5.4400x; 5.4400x over previous
import functools

import jax
import jax.numpy as jnp
from jax.experimental import pallas as pl
from jax.experimental.pallas import tpu as pltpu

F32 = jnp.float32
B, T, H, W = 2, 4, 24, 24
DIM = 192
L = T * H * W
KEEP = L // 2
NUM_RAND = int(KEEP * 0.1)
NUM_TOP = KEEP - NUM_RAND
D_INNER = 384
D_STATE = 16
DT_RANK = 12
MLP_HID = 768
CI = 256
CK = 128
HW = H * W
SCHUNK = 64

_ONES11 = None


def _iota(shape, dim):
    return jax.lax.broadcasted_iota(jnp.int32, shape, dim).astype(F32)


def _dot(a, b, dims):
    return jax.lax.dot_general(
        a, b, (dims, ((), ())),
        precision=jax.lax.Precision.HIGHEST,
        preferred_element_type=F32)


def _row_of(col):
    return _dot(jnp.ones((1, 1), F32), col, ((1,), (1,)))


def _col_of(row):
    return _dot(row, jnp.ones((1, 1), F32), ((0,), (1,)))


def _rmsnorm(x, w_row):
    return x * jax.lax.rsqrt(
        jnp.mean(x * x, axis=1, keepdims=True) + 1e-5) * w_row


def _ranks(v_col, v_row):
    j_row = _iota((1, L), 1)
    outs = []
    for c in range(L // CI):
        i_col = _iota((CI, 1), 0) + (c * CI)
        vi = v_col[c * CI:(c + 1) * CI, :]
        gt = (v_row > vi).astype(F32)
        tie = ((v_row == vi) & (j_row < i_col)).astype(F32)
        outs.append(jnp.sum(gt + tie, axis=1, keepdims=True))
    return jnp.concatenate(outs, axis=0)


def _selection(xn, r_row):
    sq_col = jnp.sqrt(jnp.sum(xn * xn, axis=1, keepdims=True))
    sq_row = _row_of(sq_col)
    rank_s = _ranks(sq_col, sq_row)
    selt_col = (rank_s < float(NUM_TOP)).astype(F32)
    selt_row = _row_of(selt_col)
    rm_row = jnp.where(selt_row > 0.5, jnp.full_like(r_row, -1e30), r_row)
    rm_col = _col_of(rm_row)
    rank_r = _ranks(rm_col, rm_row)
    selr_col = (rank_r < float(NUM_RAND)).astype(F32)
    sel_col = jnp.maximum(selt_col, selr_col)
    sel_row = _row_of(sel_col)
    j_row = _iota((1, L), 1)
    pos_outs = []
    for c in range(L // CI):
        i_col = _iota((CI, 1), 0) + (c * CI)
        pos_outs.append(jnp.sum(sel_row * (j_row < i_col).astype(F32),
                                axis=1, keepdims=True))
    pos_col = jnp.concatenate(pos_outs, axis=0)
    pos_row = _row_of(pos_col)
    xs_chunks = []
    for kc in range(KEEP // CK):
        k_col = _iota((CK, 1), 0) + (kc * CK)
        P = ((pos_row == k_col) & (sel_row > 0.5)).astype(F32)
        xs_chunks.append(_dot(P, xn, ((1,), (0,))))
    xs = jnp.concatenate(xs_chunks, axis=0)
    return xs, sel_row, pos_row


def _scatter_add(base, y, sel_row, pos_row):
    sel_col = _col_of(sel_row)
    pos_col = _col_of(pos_row)
    k_row = _iota((1, KEEP), 1)
    outs = []
    for c in range(L // CI):
        pc = pos_col[c * CI:(c + 1) * CI, :]
        sc = sel_col[c * CI:(c + 1) * CI, :]
        PT = ((k_row == pc) & (sc > 0.5)).astype(F32)
        outs.append(base[c * CI:(c + 1) * CI, :] + _dot(PT, y, ((1,), (0,))))
    return jnp.concatenate(outs, axis=0)


def _k1(x_ref, w1_ref, r_ref, xs_ref, sel_ref, pos_ref):
    x = x_ref[0]
    xn = _rmsnorm(x, w1_ref[...])
    xs, sel_row, pos_row = _selection(xn, r_ref[0])
    xs_ref[0] = xs
    sel_ref[0] = sel_row
    pos_ref[0] = pos_row


def _k2(xs_ref, inproj_ref, convwT_ref, convb_ref, xproj_ref, dtw_ref,
        dtb_ref, alogflat_ref, emat_ref, d_ref, outw_ref, y_ref, *scr):
    NC = KEEP // SCHUNK
    LA = D_STATE * D_INNER
    a_flat = -jnp.exp(alogflat_ref[...])
    emat = emat_ref[...]
    z_all = []
    for bi in range(B):
        xs = xs_ref[bi]
        xz = _dot(xs, inproj_ref[...], ((1,), (1,)))
        x_rows = xz[:, :D_INNER]
        z_all.append(xz[:, D_INNER:])
        zeros3 = jnp.zeros((3, D_INNER), F32)
        pf = jnp.concatenate([zeros3, x_rows], axis=0)
        pr = jnp.concatenate([x_rows, zeros3], axis=0)
        accf = jnp.zeros((KEEP, D_INNER), F32)
        accr = jnp.zeros((KEEP, D_INNER), F32)
        for k in range(4):
            wk = convwT_ref[k:k + 1, :]
            accf = accf + pf[k:k + KEEP, :] * wk
            accr = accr + pr[3 - k:3 - k + KEEP, :] * wk
        for di, acc in ((0, accf), (1, accr)):
            s = bi * 2 + di
            dt_s, g_s, xc_s, ys_s, bm_s, cm_s = scr[6 * s:6 * s + 6]
            xc = jax.nn.silu(acc + convb_ref[...])
            xdbl = _dot(xc, xproj_ref[...], ((1,), (1,)))
            dt = jax.nn.softplus(
                _dot(xdbl[:, :DT_RANK], dtw_ref[...], ((1,), (1,)))
                + dtb_ref[...])
            dt_s[...] = dt
            g_s[...] = dt * xc
            xc_s[...] = xc
            bm_s[...] = xdbl[:, DT_RANK:DT_RANK + D_STATE]
            cm_s[...] = xdbl[:, DT_RANK + D_STATE:]

    def body(n, hins):
        new_hins = []
        for s in range(4):
            dt_s, g_s, xc_s, ys_s, bm_s, cm_s = scr[6 * s:6 * s + 6]
            rev = (s % 2) == 1
            off = (NC - 1 - n) * SCHUNK if rev else n * SCHUNK
            dt_c = dt_s[pl.ds(off, SCHUNK), :]
            g_c = g_s[pl.ds(off, SCHUNK), :]
            bm_c = bm_s[pl.ds(off, SCHUNK), :]
            cm_c = cm_s[pl.ds(off, SCHUNK), :]
            a = jnp.exp(jnp.concatenate([dt_c] * D_STATE, 1) * a_flat)
            bb = jnp.concatenate([g_c] * D_STATE, 1) * _dot(
                bm_c, emat, ((1,), (0,)))
            k = 1
            while k < SCHUNK:
                ones_p = jnp.ones((k, LA), F32)
                zero_p = jnp.zeros((k, LA), F32)
                if rev:
                    a_sh = jnp.concatenate([a[k:], ones_p], 0)
                    b_sh = jnp.concatenate([bb[k:], zero_p], 0)
                else:
                    a_sh = jnp.concatenate([ones_p, a[:SCHUNK - k]], 0)
                    b_sh = jnp.concatenate([zero_p, bb[:SCHUNK - k]], 0)
                bb = a * b_sh + bb
                a = a * a_sh
                k *= 2
            h = a * hins[s] + bb
            new_hins.append(h[0:1] if rev else h[SCHUNK - 1:SCHUNK])
            hc = h * _dot(cm_c, emat, ((1,), (0,)))
            y_c = hc[:, 0:D_INNER]
            for q in range(1, D_STATE):
                y_c = y_c + hc[:, q * D_INNER:(q + 1) * D_INNER]
            ys_s[pl.ds(off, SCHUNK), :] = y_c
        return tuple(new_hins)

    h0 = jnp.zeros((1, LA), F32)
    jax.lax.fori_loop(0, NC, body, (h0, h0, h0, h0))

    for bi in range(B):
        _, _, xcf, ysf, _, _ = scr[6 * (2 * bi):6 * (2 * bi) + 6]
        _, _, xcr, ysr, _, _ = scr[6 * (2 * bi + 1):6 * (2 * bi + 1) + 6]
        y_rows = ysf[...] + ysr[...] + (xcf[...] + xcr[...]) * d_ref[...]
        y_rows = y_rows * jax.nn.silu(z_all[bi])
        y_ref[bi] = _dot(y_rows, outw_ref[...], ((1,), (1,)))


def _k3(x_ref, y_ref, sel1_ref, pos1_ref, w2_ref, r2_ref,
        x2_ref, xs2_ref, sel2_ref, pos2_ref):
    x2 = _scatter_add(x_ref[0], y_ref[0], sel1_ref[0], pos1_ref[0])
    x2_ref[0] = x2
    xn2 = _rmsnorm(x2, w2_ref[...])
    xs2, sel_row, pos_row = _selection(xn2, r2_ref[0])
    xs2_ref[0] = xs2
    sel2_ref[0] = sel_row
    pos2_ref[0] = pos_row


def _k4a(xs2_ref, sel2_ref, pos2_ref, fc1_ref, fc1b_ref, dww_ref,
         dwb_ref, hd2_ref, hd_s):
    h1 = _dot(xs2_ref[0], fc1_ref[...], ((1,), (1,))) + fc1b_ref[...]
    sel_row = sel2_ref[0]
    pos_row = pos2_ref[0]
    sel_col = _col_of(sel_row)
    pos_col = _col_of(pos_row)
    k_row = _iota((1, KEEP), 1)
    for c in range(L // CI):
        pc = pos_col[c * CI:(c + 1) * CI, :]
        sc = sel_col[c * CI:(c + 1) * CI, :]
        PT = ((k_row == pc) & (sc > 0.5)).astype(F32)
        hd_s[c * CI:(c + 1) * CI, :] = _dot(PT, h1, ((1,), (0,)))
    xpos = jnp.remainder(_iota((HW, 1), 0), 24.0)
    for f in range(T):
        img = hd_s[f * HW:(f + 1) * HW, :]
        zp = jnp.zeros((32, MLP_HID), F32)
        pb = jnp.concatenate([zp, img, zp], axis=0)
        acc = jnp.zeros((HW, MLP_HID), F32)
        for dy in (-1, 0, 1):
            for dx in (-1, 0, 1):
                sh = 32 + dy * 24 + dx
                term = pb[sh:sh + HW, :] * dww_ref[(dy + 1) * 3 + dx + 1:
                                                   (dy + 1) * 3 + dx + 2, :]
                if dx == -1:
                    term = term * (xpos >= 1.0).astype(F32)
                elif dx == 1:
                    term = term * (xpos <= 22.0).astype(F32)
                acc = acc + term
        hd2_ref[0, f * HW:(f + 1) * HW, :] = acc + dwb_ref[...]


def _k4b(hd2_ref, sel2_ref, pos2_ref, x2_ref, fc2_ref, fc2b_ref, out_ref):
    sel_row = sel2_ref[0]
    pos_row = pos2_ref[0]
    hd2 = hd2_ref[0]
    mo_chunks = []
    for kc in range(KEEP // CK):
        k_col = _iota((CK, 1), 0) + (kc * CK)
        P = ((pos_row == k_col) & (sel_row > 0.5)).astype(F32)
        xg = _dot(P, hd2, ((1,), (0,)))
        xg = 0.5 * xg * (1.0 + jax.lax.erf(xg * 0.7071067811865476))
        mo_chunks.append(_dot(xg, fc2_ref[...], ((1,), (1,))) + fc2b_ref[...])
    mo = jnp.concatenate(mo_chunks, axis=0)
    out_ref[0] = _scatter_add(x2_ref[0], mo, sel_row, pos_row)


def _bs(shape, imap):
    return pl.BlockSpec(shape, imap)


def _full(shape):
    n = len(shape)
    return pl.BlockSpec(shape, lambda b: (0,) * n)


def kernel(x_in, norm1_w, norm2_w, in_proj_w, conv_w, conv_b, x_proj_w,
           dt_proj_w, dt_proj_b, A_log, D_param, out_proj_w, fc1_w, fc1_b,
           dw_w, dw_b, fc2_w, fc2_b):
    x_flat = jnp.transpose(x_in, (0, 1, 3, 4, 2)).reshape(B, L, DIM)
    r1 = jax.random.uniform(jax.random.key(42), (B, L)).reshape(B, 1, L)
    r2 = jax.random.uniform(jax.random.key(43), (B, L)).reshape(B, 1, L)
    w1 = norm1_w.reshape(1, DIM)
    w2 = norm2_w.reshape(1, DIM)
    convwT = jnp.transpose(conv_w)
    convb = conv_b.reshape(1, D_INNER)
    dtb = dt_proj_b.reshape(1, D_INNER)
    dvec = D_param.reshape(1, D_INNER)
    fc1b = fc1_b.reshape(1, MLP_HID)
    dww = dw_w.reshape(9, MLP_HID)
    dwb = dw_b.reshape(1, MLP_HID)
    fc2b = fc2_b.reshape(1, DIM)

    sd = jax.ShapeDtypeStruct
    xs1, sel1, pos1 = pl.pallas_call(
        _k1,
        grid=(B,),
        in_specs=[_bs((1, L, DIM), lambda b: (b, 0, 0)),
                  _full((1, DIM)),
                  _bs((1, 1, L), lambda b: (b, 0, 0))],
        out_specs=[_bs((1, KEEP, DIM), lambda b: (b, 0, 0)),
                   _bs((1, 1, L), lambda b: (b, 0, 0)),
                   _bs((1, 1, L), lambda b: (b, 0, 0))],
        out_shape=[sd((B, KEEP, DIM), F32),
                   sd((B, 1, L), F32),
                   sd((B, 1, L), F32)],
    )(x_flat, w1, r1)

    alogflat = jnp.transpose(A_log).reshape(1, D_STATE * D_INNER)
    emat = jnp.kron(jnp.eye(D_STATE, dtype=F32),
                    jnp.ones((1, D_INNER), F32))
    scratch = []
    for _ in range(4):
        scratch += [pltpu.VMEM((KEEP, D_INNER), F32),
                    pltpu.VMEM((KEEP, D_INNER), F32),
                    pltpu.VMEM((KEEP, D_INNER), F32),
                    pltpu.VMEM((KEEP, D_INNER), F32),
                    pltpu.VMEM((KEEP, D_STATE), F32),
                    pltpu.VMEM((KEEP, D_STATE), F32)]
    y = pl.pallas_call(
        _k2,
        grid=(1,),
        in_specs=[_full((B, KEEP, DIM)),
                  _full((2 * D_INNER, DIM)),
                  _full((4, D_INNER)),
                  _full((1, D_INNER)),
                  _full((DT_RANK + 2 * D_STATE, D_INNER)),
                  _full((D_INNER, DT_RANK)),
                  _full((1, D_INNER)),
                  _full((1, D_STATE * D_INNER)),
                  _full((D_STATE, D_STATE * D_INNER)),
                  _full((1, D_INNER)),
                  _full((DIM, D_INNER))],
        out_specs=_full((B, KEEP, DIM)),
        out_shape=sd((B, KEEP, DIM), F32),
        scratch_shapes=scratch,
    )(xs1, in_proj_w, convwT, convb, x_proj_w, dt_proj_w, dtb, alogflat,
      emat, dvec, out_proj_w)

    x2, xs2, sel2, pos2 = pl.pallas_call(
        _k3,
        grid=(B,),
        in_specs=[_bs((1, L, DIM), lambda b: (b, 0, 0)),
                  _bs((1, KEEP, DIM), lambda b: (b, 0, 0)),
                  _bs((1, 1, L), lambda b: (b, 0, 0)),
                  _bs((1, 1, L), lambda b: (b, 0, 0)),
                  _full((1, DIM)),
                  _bs((1, 1, L), lambda b: (b, 0, 0))],
        out_specs=[_bs((1, L, DIM), lambda b: (b, 0, 0)),
                   _bs((1, KEEP, DIM), lambda b: (b, 0, 0)),
                   _bs((1, 1, L), lambda b: (b, 0, 0)),
                   _bs((1, 1, L), lambda b: (b, 0, 0))],
        out_shape=[sd((B, L, DIM), F32),
                   sd((B, KEEP, DIM), F32),
                   sd((B, 1, L), F32),
                   sd((B, 1, L), F32)],
    )(x_flat, y, sel1, pos1, w2, r2)

    hd2 = pl.pallas_call(
        _k4a,
        grid=(B,),
        in_specs=[_bs((1, KEEP, DIM), lambda b: (b, 0, 0)),
                  _bs((1, 1, L), lambda b: (b, 0, 0)),
                  _bs((1, 1, L), lambda b: (b, 0, 0)),
                  _full((MLP_HID, DIM)),
                  _full((1, MLP_HID)),
                  _full((9, MLP_HID)),
                  _full((1, MLP_HID))],
        out_specs=_bs((1, L, MLP_HID), lambda b: (b, 0, 0)),
        out_shape=sd((B, L, MLP_HID), F32),
        scratch_shapes=[pltpu.VMEM((L, MLP_HID), F32)],
    )(xs2, sel2, pos2, fc1_w, fc1b, dww, dwb)

    out = pl.pallas_call(
        _k4b,
        grid=(B,),
        in_specs=[_bs((1, L, MLP_HID), lambda b: (b, 0, 0)),
                  _bs((1, 1, L), lambda b: (b, 0, 0)),
                  _bs((1, 1, L), lambda b: (b, 0, 0)),
                  _bs((1, L, DIM), lambda b: (b, 0, 0)),
                  _full((DIM, MLP_HID)),
                  _full((1, DIM))],
        out_specs=_bs((1, L, DIM), lambda b: (b, 0, 0)),
        out_shape=sd((B, L, DIM), F32),
    )(hd2, sel2, pos2, x2, fc2_w, fc2b)
    return out
